# baseline (speedup 1.0000x reference)
import functools

import jax
import jax.numpy as jnp
from jax import lax
from jax.experimental import pallas as pl
from jax.experimental.pallas import tpu as pltpu
from jax.experimental.pallas import tpu_sc as plsc

_CHUNK = 640
_SUB = 128


@functools.cache
def _build_gather(B, V, D):
    info = plsc.get_sparse_core_info()
    NC, NS = info.num_cores, info.num_subcores
    NW = NC * NS
    b_per_w = B // NW
    assert B % NW == 0 and b_per_w % _CHUNK == 0 and _CHUNK % _SUB == 0
    n_chunks = b_per_w // _CHUNK
    assert n_chunks % 2 == 0
    n_sub = _CHUNK // _SUB
    mesh = plsc.VectorSubcoreMesh(core_axis_name="c", subcore_axis_name="s")

    @functools.partial(
        pl.kernel,
        out_type=jax.ShapeDtypeStruct((B, D), jnp.float32),
        mesh=mesh,
        scratch_types=[
            pltpu.VMEM((b_per_w,), jnp.int32),
            pltpu.VMEM((2, _CHUNK, D), jnp.float32),
            pltpu.SemaphoreType.DMA,
            pltpu.SemaphoreType.DMA,
        ],
        compiler_params=pltpu.CompilerParams(use_tc_tiling_on_sc=False),
    )
    def gather_kernel(idx_hbm, table_hbm, out_hbm, idx_v, rows_v, sem0, sem1):
        sems = (sem0, sem1)
        wid = lax.axis_index("s") * NC + lax.axis_index("c")
        base = wid * b_per_w
        pltpu.sync_copy(idx_hbm.at[pl.ds(base, b_per_w)], idx_v)

        def fire(c, buf):
            for j in range(n_sub):
                pltpu.async_copy(
                    table_hbm.at[idx_v.at[pl.ds(c * _CHUNK + j * _SUB, _SUB)]],
                    rows_v.at[buf].at[pl.ds(j * _SUB, _SUB)],
                    sems[buf],
                )

        def drain(buf):
            pltpu.make_async_copy(
                table_hbm.at[pl.ds(0, _CHUNK)], rows_v.at[buf], sems[buf]
            ).wait()

        fire(0, 0)

        def body(it, carry):
            i = it * 2
            for buf in range(2):
                c = i + buf

                @pl.when(c + 1 < n_chunks)
                def _():
                    fire(c + 1, 1 - buf)

                drain(buf)
                pltpu.sync_copy(
                    rows_v.at[buf],
                    out_hbm.at[pl.ds(base + c * _CHUNK, _CHUNK)],
                )
            return carry

        lax.fori_loop(0, n_chunks // 2, body, 0)

    return gather_kernel


def kernel(x, table):
    b, t = x.shape
    V, D = table.shape
    flat = jnp.reshape(x, (-1,)).astype(jnp.int32)
    out = _build_gather(b * t, V, D)(flat, table)
    return jnp.reshape(out, (b, t, D))
